# baseline (speedup 1.0000x reference)
import functools

import jax
import jax.numpy as jnp
from jax import lax
from jax.experimental import pallas as pl
from jax.experimental.pallas import tpu as pltpu
from jax.experimental.pallas import tpu_sc as plsc

N1 = 50000
D = 256
N2 = 12500
K = 16
L = 16

NC, NS = 2, 16
NW = NC * NS
ROWS_PER_W = -(-N2 // NW)

_SHADOW_BLK = 2000


def _shadow_body(x_ref, o_ref):
    i = pl.program_id(0)
    m = jnp.min(x_ref[...], axis=0, keepdims=True)

    @pl.when(i == 0)
    def _():
        o_ref[...] = m

    @pl.when(i > 0)
    def _():
        o_ref[...] = jnp.minimum(o_ref[...], m)


def _col_min(x):
    return pl.pallas_call(
        _shadow_body,
        grid=(N1 // _SHADOW_BLK,),
        in_specs=[pl.BlockSpec((_SHADOW_BLK, D), lambda i: (i, 0))],
        out_specs=pl.BlockSpec((1, D), lambda i: (0, 0)),
        out_shape=jax.ShapeDtypeStruct((1, D), jnp.float32),
    )(x)


@functools.partial(
    pl.kernel,
    out_type=jax.ShapeDtypeStruct((N2, D), jnp.float32),
    mesh=plsc.VectorSubcoreMesh(core_axis_name="c", subcore_axis_name="s"),
    scratch_types=[
        pltpu.VMEM((K,), jnp.int32),
        pltpu.VMEM((K, D), jnp.float32),
        pltpu.VMEM((D,), jnp.float32),
        pltpu.VMEM((D,), jnp.float32),
        pltpu.SemaphoreType.DMA,
    ],
)
def _sc_pool(x_hbm, inds_hbm, shadow_hbm, out_hbm,
             idx_v, gbuf, obuf, shadow_v, sem):
    c = lax.axis_index("c")
    s = lax.axis_index("s")
    wid = s * NC + c
    base = wid * ROWS_PER_W

    pltpu.sync_copy(shadow_hbm, shadow_v)

    def row_step(i, carry):
        row = base + i

        @pl.when(row < N2)
        def _():
            pltpu.sync_copy(inds_hbm.at[row], idx_v)
            idx = idx_v[...]
            lane = lax.iota(jnp.int32, L)
            minvec = idx
            for sh in (8, 4, 2, 1):
                perm = (lane + sh) & (L - 1)
                minvec = jnp.minimum(
                    minvec, minvec.at[perm].get(mode="promise_in_bounds"))
            remapped = jnp.where(idx == N1, minvec, idx)
            idx_v[...] = jnp.minimum(remapped, N1 - 1)
            allinv = minvec == N1
            pltpu.async_copy(x_hbm.at[idx_v], gbuf, sem).wait()
            for cc in range(D // L):
                sl = pl.ds(cc * L, L)
                a = gbuf[0, sl]
                for j in range(1, K):
                    a = jnp.maximum(a, gbuf[j, sl])
                obuf[sl] = jnp.where(allinv, shadow_v[sl], a)
            pltpu.sync_copy(obuf, out_hbm.at[row])

        return carry

    lax.fori_loop(0, ROWS_PER_W, row_step, 0)


def kernel(x, inds):
    shadow = _col_min(x)
    return _sc_pool(x, inds.astype(jnp.int32), shadow.reshape(D))


# speedup vs baseline: 4.0441x; 4.0441x over previous
import functools

import jax
import jax.numpy as jnp
from jax import lax
from jax.experimental import pallas as pl
from jax.experimental.pallas import tpu as pltpu
from jax.experimental.pallas import tpu_sc as plsc

N1 = 50000
D = 256
N2 = 12500
K = 16
L = 16

NC, NS = 2, 16
NW = NC * NS
B = 8
NB = 50
WROWS = B * NB
N2P = NW * WROWS

_SHADOW_BLK = 2000


def _shadow_body(x_ref, o_ref):
    i = pl.program_id(0)
    m = jnp.min(x_ref[...], axis=0, keepdims=True)

    @pl.when(i == 0)
    def _():
        o_ref[...] = m

    @pl.when(i > 0)
    def _():
        o_ref[...] = jnp.minimum(o_ref[...], m)


def _col_min(x):
    return pl.pallas_call(
        _shadow_body,
        grid=(N1 // _SHADOW_BLK,),
        in_specs=[pl.BlockSpec((_SHADOW_BLK, D), lambda i: (i, 0))],
        out_specs=pl.BlockSpec((1, D), lambda i: (0, 0)),
        out_shape=jax.ShapeDtypeStruct((1, D), jnp.float32),
    )(x)


def _lane_min_all(idx, lane):
    m = idx
    for sh in (8, 4, 2, 1):
        perm = (lane + sh) & (L - 1)
        m = jnp.minimum(m, m.at[perm].get(mode="promise_in_bounds"))
    return m


@functools.partial(
    pl.kernel,
    out_type=jax.ShapeDtypeStruct((N2, D), jnp.float32),
    mesh=plsc.VectorSubcoreMesh(core_axis_name="c", subcore_axis_name="s"),
    scratch_types=[
        pltpu.VMEM((WROWS, K), jnp.int32),
        pltpu.VMEM((B * K,), jnp.int32),
        pltpu.VMEM((B * K,), jnp.int32),
        pltpu.VMEM((B * K, D), jnp.float32),
        pltpu.VMEM((B * K, D), jnp.float32),
        pltpu.VMEM((B, D), jnp.float32),
        pltpu.VMEM((D,), jnp.float32),
        pltpu.SemaphoreType.DMA,
        pltpu.SemaphoreType.DMA,
    ],
)
def _sc_pool(x_hbm, inds_hbm, shadow_hbm, out_hbm,
             idx_all, gidx0, gidx1, gbuf0, gbuf1, obuf, shadow_v,
             sem0, sem1):
    c = lax.axis_index("c")
    s = lax.axis_index("s")
    wid = s * NC + c
    base = wid * WROWS

    pltpu.sync_copy(shadow_hbm, shadow_v)
    pltpu.sync_copy(inds_hbm.at[pl.ds(base, WROWS)], idx_all)

    lane = lax.iota(jnp.int32, L)

    def live(t):
        return base + t * B < N2

    def prep(t, gidx):
        for r in range(B):
            idx = idx_all[t * B + r, :]
            minvec = _lane_min_all(idx, lane)
            remapped = jnp.where(idx == N1, minvec, idx)
            gidx[pl.ds(r * K, K)] = jnp.minimum(remapped, N1 - 1)

    def compute_store(t, gbuf):
        def crow(r, carry):
            idx = idx_all[t * B + r, :]
            minvec = _lane_min_all(idx, lane)
            allinv = minvec == N1
            for cc in range(D // L):
                sl = pl.ds(cc * L, L)
                a = gbuf[r * K, sl]
                for j in range(1, K):
                    a = jnp.maximum(a, gbuf[r * K + j, sl])
                obuf[r, sl] = jnp.where(allinv, shadow_v[sl], a)
            return carry

        lax.fori_loop(0, B, crow, 0)
        row0 = base + t * B

        @pl.when(row0 + B <= N2)
        def _():
            pltpu.sync_copy(obuf, out_hbm.at[pl.ds(row0, B)])

        @pl.when(row0 + B > N2)
        def _():
            for r in range(B):
                @pl.when(row0 + r < N2)
                def _():
                    pltpu.sync_copy(obuf.at[r], out_hbm.at[row0 + r])

    @pl.when(live(0))
    def _():
        prep(0, gidx0)
        pltpu.async_copy(x_hbm.at[gidx0], gbuf0, sem0)

    @pl.when(live(1))
    def _():
        prep(1, gidx1)
        pltpu.async_copy(x_hbm.at[gidx1], gbuf1, sem1)

    def outer(g, carry):
        t0 = 2 * g
        t1 = 2 * g + 1

        @pl.when(live(t0))
        def _():
            pltpu.make_async_copy(x_hbm.at[gidx0], gbuf0, sem0).wait()
            compute_store(t0, gbuf0)

        @pl.when(live(t0 + 2) & (t0 + 2 < NB))
        def _():
            prep(t0 + 2, gidx0)
            pltpu.async_copy(x_hbm.at[gidx0], gbuf0, sem0)

        @pl.when(live(t1))
        def _():
            pltpu.make_async_copy(x_hbm.at[gidx1], gbuf1, sem1).wait()
            compute_store(t1, gbuf1)

        @pl.when(live(t1 + 2) & (t1 + 2 < NB))
        def _():
            prep(t1 + 2, gidx1)
            pltpu.async_copy(x_hbm.at[gidx1], gbuf1, sem1)

        return carry

    lax.fori_loop(0, NB // 2, outer, 0)


def kernel(x, inds):
    shadow = _col_min(x)
    inds32 = jnp.pad(inds.astype(jnp.int32), ((0, N2P - N2), (0, 0)))
    return _sc_pool(x, inds32, shadow.reshape(D))


# speedup vs baseline: 4.2143x; 1.0421x over previous
import functools

import jax
import jax.numpy as jnp
from jax import lax
from jax.experimental import pallas as pl
from jax.experimental.pallas import tpu as pltpu
from jax.experimental.pallas import tpu_sc as plsc

N1 = 50000
D = 256
N2 = 12500
K = 16
L = 16

NC, NS = 2, 16
NW = NC * NS
B = 8
NB = 50
WROWS = B * NB
N2P = NW * WROWS

_SHADOW_BLK = 2000


def _shadow_body(x_ref, o_ref):
    i = pl.program_id(0)
    m = jnp.min(x_ref[...], axis=0, keepdims=True)

    @pl.when(i == 0)
    def _():
        o_ref[...] = m

    @pl.when(i > 0)
    def _():
        o_ref[...] = jnp.minimum(o_ref[...], m)


def _col_min(x):
    return pl.pallas_call(
        _shadow_body,
        grid=(N1 // _SHADOW_BLK,),
        in_specs=[pl.BlockSpec((_SHADOW_BLK, D), lambda i: (i, 0))],
        out_specs=pl.BlockSpec((1, D), lambda i: (0, 0)),
        out_shape=jax.ShapeDtypeStruct((1, D), jnp.float32),
    )(x)


def _lane_min_all(idx, lane):
    m = idx
    for sh in (8, 4, 2, 1):
        perm = (lane + sh) & (L - 1)
        m = jnp.minimum(m, m.at[perm].get(mode="promise_in_bounds"))
    return m


@functools.partial(
    pl.kernel,
    out_type=(
        jax.ShapeDtypeStruct((N2, D), jnp.float32),
        jax.ShapeDtypeStruct((NW * 56, K), jnp.int32),
    ),
    mesh=plsc.VectorSubcoreMesh(core_axis_name="c", subcore_axis_name="s"),
    scratch_types=[
        pltpu.VMEM((WROWS, K), jnp.int32),
        pltpu.VMEM((56, K), jnp.int32),
        pltpu.VMEM((B * K,), jnp.int32),
        pltpu.VMEM((B * K,), jnp.int32),
        pltpu.VMEM((B * K, D), jnp.float32),
        pltpu.VMEM((B * K, D), jnp.float32),
        pltpu.VMEM((B, D), jnp.float32),
        pltpu.SemaphoreType.DMA,
        pltpu.SemaphoreType.DMA,
    ],
)
def _sc_pool(x_hbm, inds_hbm, out_hbm, flags_hbm,
             idx_all, minbuf, gidx0, gidx1, gbuf0, gbuf1, obuf,
             sem0, sem1):
    c = lax.axis_index("c")
    s = lax.axis_index("s")
    wid = s * NC + c
    base = wid * WROWS

    pltpu.sync_copy(inds_hbm.at[pl.ds(base, WROWS)], idx_all)

    lane = lax.iota(jnp.int32, L)

    def live(t):
        return base + t * B < N2

    def prep(t, gidx):
        pack = jnp.zeros((L,), jnp.int32)
        for r in range(B):
            idx = idx_all[t * B + r, :]
            minvec = _lane_min_all(idx, lane)
            pack = jnp.where(lane == r, minvec, pack)
            remapped = jnp.where(idx == N1, minvec, idx)
            gidx[pl.ds(r * K, K)] = jnp.minimum(remapped, N1 - 1)
        minbuf[t, :] = pack

    def compute_store(t, gbuf):
        def crow(r, carry):
            for cc in range(D // L):
                sl = pl.ds(cc * L, L)
                a = gbuf[r * K, sl]
                for j in range(1, K):
                    a = jnp.maximum(a, gbuf[r * K + j, sl])
                obuf[r, sl] = a
            return carry

        lax.fori_loop(0, B, crow, 0)
        row0 = base + t * B

        @pl.when(row0 + B <= N2)
        def _():
            pltpu.sync_copy(obuf, out_hbm.at[pl.ds(row0, B)])

        @pl.when(row0 + B > N2)
        def _():
            for r in range(B):
                @pl.when(row0 + r < N2)
                def _():
                    pltpu.sync_copy(obuf.at[r], out_hbm.at[row0 + r])

    @pl.when(live(0))
    def _():
        prep(0, gidx0)
        pltpu.async_copy(x_hbm.at[gidx0], gbuf0, sem0)

    @pl.when(live(1))
    def _():
        prep(1, gidx1)
        pltpu.async_copy(x_hbm.at[gidx1], gbuf1, sem1)

    def outer(g, carry):
        t0 = 2 * g
        t1 = 2 * g + 1

        @pl.when(live(t0))
        def _():
            pltpu.make_async_copy(x_hbm.at[gidx0], gbuf0, sem0).wait()
            compute_store(t0, gbuf0)

        @pl.when(live(t0 + 2) & (t0 + 2 < NB))
        def _():
            prep(t0 + 2, gidx0)
            pltpu.async_copy(x_hbm.at[gidx0], gbuf0, sem0)

        @pl.when(live(t1))
        def _():
            pltpu.make_async_copy(x_hbm.at[gidx1], gbuf1, sem1).wait()
            compute_store(t1, gbuf1)

        @pl.when(live(t1 + 2) & (t1 + 2 < NB))
        def _():
            prep(t1 + 2, gidx1)
            pltpu.async_copy(x_hbm.at[gidx1], gbuf1, sem1)

        return carry

    lax.fori_loop(0, NB // 2, outer, 0)
    pltpu.sync_copy(minbuf, flags_hbm.at[pl.ds(wid * 56, 56)])


def kernel(x, inds):
    shadow = _col_min(x)
    inds32 = jnp.pad(inds.astype(jnp.int32), ((0, N2P - N2), (0, 0)))
    sc_out, flags = _sc_pool(x, inds32)
    rowmin = flags.reshape(NW, 56, K)[:, :NB, :B].reshape(N2P)[:N2]
    return jnp.where((rowmin == N1)[:, None], shadow, sc_out)
